# baseline (speedup 1.0000x reference)
import functools

import jax
import jax.numpy as jnp
from jax import lax
from jax.experimental import pallas as pl
from jax.experimental.pallas import tpu as pltpu
from jax.experimental.pallas import tpu_sc as plsc

N = 10000
E = 320000
D = 128
H = 128
C = 16
G = 64

NW = 32
CH = 80
NG = 5
CPG = 25
NP = 10240
RPS = NP // 16


def _segment_sum_sc(h, src_r, dst_r):
    mesh = plsc.VectorSubcoreMesh(core_axis_name="c", subcore_axis_name="s")

    @functools.partial(
        pl.kernel,
        mesh=mesh,
        out_type=jax.ShapeDtypeStruct((2, NP, D), jnp.float32),
        scratch_types=[
            pltpu.VMEM((CPG, CH), jnp.int32),
            pltpu.VMEM((CPG, CH), jnp.int32),
            pltpu.VMEM((CPG, CH), jnp.int32),
            pltpu.VMEM((CPG, CH), jnp.int32),
            pltpu.VMEM((CH, D), jnp.float32),
            pltpu.VMEM((CH, D), jnp.float32),
            pltpu.VMEM((CH, D), jnp.float32),
            pltpu.VMEM_SHARED((NP, D), jnp.float32),
            pltpu.SemaphoreType.DMA,
            pltpu.SemaphoreType.DMA,
            pltpu.SemaphoreType.DMA,
            pltpu.SemaphoreType.DMA,
            pltpu.SemaphoreType.DMA,
        ],
    )
    def agg(h_hbm, src_hbm, dst_hbm, out_hbm,
            sv0, dv0, sv1, dv1, b0, b1, b2, acc,
            s0, s1, s2, si0, si1):
        cid = lax.axis_index("c")
        sid = lax.axis_index("s")
        wid = sid * 2 + cid

        idxb = ((sv0, dv0, si0), (sv1, dv1, si1))

        def idx_start(g):
            sv, dv, si = idxb[g % 2]
            pltpu.async_copy(src_hbm.at[wid, g], sv, si)
            pltpu.async_copy(dst_hbm.at[wid, g], dv, si)

        def idx_wait(g):
            sv, dv, si = idxb[g % 2]
            pltpu.make_async_copy(src_hbm.at[wid, g], sv, si).wait()
            pltpu.make_async_copy(dst_hbm.at[wid, g], dv, si).wait()

        bufs = ((b0, s0), (b1, s1), (b2, s2))

        def issue(g, j, b):
            pltpu.async_copy(
                h_hbm.at[idxb[g % 2][0].at[j]], bufs[b][0], bufs[b][1])

        def proc(g, j, b):
            pltpu.make_async_copy(
                h_hbm.at[idxb[g % 2][0].at[j]], bufs[b][0], bufs[b][1]).wait()
            pltpu.sync_copy(bufs[b][0], acc.at[idxb[g % 2][1].at[j]], add=True)

        idx_start(0)

        def zrow(r, carry):
            for c in range(D // 16):
                b2[r, pl.ds(c * 16, 16)] = jnp.zeros((16,), jnp.float32)
            return carry

        lax.fori_loop(0, CH, zrow, 0)
        idx_wait(0)
        issue(0, 0, 0)
        issue(0, 1, 1)
        idx_start(1)
        for k in range(RPS // CH):
            pltpu.sync_copy(b2, acc.at[pl.ds(sid * RPS + k * CH, CH)])
        plsc.subcore_barrier()

        for g in range(NG):

            def body(i, carry, g=g):
                j = 3 * i
                issue(g, j + 2, 2)
                proc(g, j, 0)
                issue(g, j + 3, 0)
                proc(g, j + 1, 1)
                issue(g, j + 4, 1)
                proc(g, j + 2, 2)
                return carry

            lax.fori_loop(0, (CPG - 4) // 3, body, 0)
            issue(g, CPG - 2, 2)
            proc(g, CPG - 4, 0)
            issue(g, CPG - 1, 0)
            proc(g, CPG - 3, 1)
            proc(g, CPG - 2, 2)
            proc(g, CPG - 1, 0)

            if g + 1 < NG:
                idx_wait(g + 1)
                issue(g + 1, 0, 0)
                issue(g + 1, 1, 1)
                if g + 2 < NG:
                    idx_start(g + 2)

        plsc.subcore_barrier()
        pltpu.sync_copy(acc.at[pl.ds(sid * RPS, RPS)],
                        out_hbm.at[cid, pl.ds(sid * RPS, RPS)])

    return agg(h, src_r, dst_r)


BR = 2000


def _mlp_body(x_ref, a_ref, s_ref, w1_ref, b1_ref, w2_ref, b2_ref, o_ref):
    z = x_ref[...] * s_ref[...] + a_ref[0] + a_ref[1]
    z = jnp.maximum(
        jnp.dot(z, w1_ref[...], preferred_element_type=jnp.float32)
        + b1_ref[...], 0.0)
    z = jnp.maximum(
        jnp.dot(z, w2_ref[...], preferred_element_type=jnp.float32)
        + b2_ref[...], 0.0)
    o_ref[...] = z


def _mlp_tc(x, a, s, W1, b1, W2, b2):
    grid = (N // BR,)
    return pl.pallas_call(
        _mlp_body,
        grid=grid,
        in_specs=[
            pl.BlockSpec((BR, D), lambda i: (i, 0)),
            pl.BlockSpec((2, BR, D), lambda i: (0, i, 0)),
            pl.BlockSpec((1, D), lambda i: (0, 0)),
            pl.BlockSpec((D, H), lambda i: (0, 0)),
            pl.BlockSpec((1, H), lambda i: (0, 0)),
            pl.BlockSpec((H, H), lambda i: (0, 0)),
            pl.BlockSpec((1, H), lambda i: (0, 0)),
        ],
        out_specs=pl.BlockSpec((BR, H), lambda i: (i, 0)),
        out_shape=jax.ShapeDtypeStruct((N, H), jnp.float32),
    )(x, a, s, W1, b1, W2, b2)


def _mlp_pool_body(x_ref, a_ref, s_ref, w1_ref, b1_ref, w2_ref, b2_ref,
                   batch_ref, wc_ref, bc_ref, o_ref, acc_s, acc_c):
    i = pl.program_id(0)
    z = x_ref[...] * s_ref[...] + a_ref[0] + a_ref[1]
    z = jnp.maximum(
        jnp.dot(z, w1_ref[...], preferred_element_type=jnp.float32)
        + b1_ref[...], 0.0)
    h2 = jnp.maximum(
        jnp.dot(z, w2_ref[...], preferred_element_type=jnp.float32)
        + b2_ref[...], 0.0)
    b = batch_ref[0]
    gid = lax.broadcasted_iota(jnp.int32, (G, BR), 0)
    p = (gid == jnp.broadcast_to(b, (G, BR))).astype(jnp.float32)

    @pl.when(i == 0)
    def _():
        acc_s[...] = jnp.zeros_like(acc_s)
        acc_c[...] = jnp.zeros_like(acc_c)

    acc_s[...] += jnp.dot(p, h2, preferred_element_type=jnp.float32)
    acc_c[...] += jnp.broadcast_to(
        jnp.sum(p, axis=1, keepdims=True), (G, H))

    @pl.when(i == pl.num_programs(0) - 1)
    def _():
        rep = acc_s[...] / jnp.maximum(acc_c[...], 1.0)
        o_ref[...] = (
            jnp.dot(rep, wc_ref[...], preferred_element_type=jnp.float32)
            + bc_ref[...])


def _mlp_pool_tc(x, a, s, W1, b1, W2, b2, batch_r, Wc_pad, bc_pad):
    grid = (N // BR,)
    return pl.pallas_call(
        _mlp_pool_body,
        grid=grid,
        in_specs=[
            pl.BlockSpec((BR, D), lambda i: (i, 0)),
            pl.BlockSpec((2, BR, D), lambda i: (0, i, 0)),
            pl.BlockSpec((1, D), lambda i: (0, 0)),
            pl.BlockSpec((D, H), lambda i: (0, 0)),
            pl.BlockSpec((1, H), lambda i: (0, 0)),
            pl.BlockSpec((H, H), lambda i: (0, 0)),
            pl.BlockSpec((1, H), lambda i: (0, 0)),
            pl.BlockSpec((1, 1, BR), lambda i: (i, 0, 0)),
            pl.BlockSpec((H, 128), lambda i: (0, 0)),
            pl.BlockSpec((1, 128), lambda i: (0, 0)),
        ],
        out_specs=pl.BlockSpec((G, 128), lambda i: (0, 0)),
        out_shape=jax.ShapeDtypeStruct((G, 128), jnp.float32),
        scratch_shapes=[
            pltpu.VMEM((G, H), jnp.float32),
            pltpu.VMEM((G, H), jnp.float32),
        ],
    )(x, a, s, W1, b1, W2, b2, batch_r, Wc_pad, bc_pad)


def kernel(x, edge_index, batch, eps0, W1_0, b1_0, W2_0, b2_0,
           eps1, W1_1, b1_1, W2_1, b2_1, Wc, bc):
    src_r = edge_index[0].reshape(NW, NG, CPG, CH)
    dst_r = edge_index[1].reshape(NW, NG, CPG, CH)
    ones_row = jnp.ones((1, D), jnp.float32)
    s0 = ones_row * (1.0 + eps0)
    s1 = ones_row * (1.0 + eps1)
    batch_r = batch.reshape(N // BR, 1, BR)
    Wc_pad = jnp.zeros((H, 128), jnp.float32).at[:, :C].set(Wc)
    bc_pad = jnp.zeros((1, 128), jnp.float32).at[0, :C].set(bc)

    a0 = _segment_sum_sc(x, src_r, dst_r)
    h1 = _mlp_tc(x, a0, s0, W1_0, b1_0.reshape(1, H), W2_0, b2_0.reshape(1, H))
    a1 = _segment_sum_sc(h1, src_r, dst_r)
    out = _mlp_pool_tc(h1, a1, s1, W1_1, b1_1.reshape(1, H),
                       W2_1, b2_1.reshape(1, H), batch_r, Wc_pad, bc_pad)
    return out[:, :C]


# speedup vs baseline: 1.0083x; 1.0083x over previous
import functools

import jax
import jax.numpy as jnp
from jax import lax
from jax.experimental import pallas as pl
from jax.experimental.pallas import tpu as pltpu
from jax.experimental.pallas import tpu_sc as plsc

N = 10000
E = 320000
D = 128
H = 128
C = 16
G = 64

NW = 32
CH = 80
NG = 5
CPG = 25
NP = 10240
RPS = NP // 16


def _segment_sum_sc(h, src_r, dst_r):
    mesh = plsc.VectorSubcoreMesh(core_axis_name="c", subcore_axis_name="s")

    @functools.partial(
        pl.kernel,
        mesh=mesh,
        out_type=jax.ShapeDtypeStruct((2, NP, D), jnp.float32),
        scratch_types=[
            pltpu.VMEM((CPG, CH), jnp.int32),
            pltpu.VMEM((CPG, CH), jnp.int32),
            pltpu.VMEM((CPG, CH), jnp.int32),
            pltpu.VMEM((CPG, CH), jnp.int32),
            pltpu.VMEM((CH, D), jnp.float32),
            pltpu.VMEM((CH, D), jnp.float32),
            pltpu.VMEM((CH, D), jnp.float32),
            pltpu.VMEM_SHARED((NP, D), jnp.float32),
            pltpu.SemaphoreType.DMA,
            pltpu.SemaphoreType.DMA,
            pltpu.SemaphoreType.DMA,
            pltpu.SemaphoreType.DMA,
            pltpu.SemaphoreType.DMA,
        ],
    )
    def agg(h_hbm, src_hbm, dst_hbm, out_hbm,
            sv0, dv0, sv1, dv1, b0, b1, b2, acc,
            s0, s1, s2, si0, si1):
        cid = lax.axis_index("c")
        sid = lax.axis_index("s")
        wid = sid * 2 + cid

        idxb = ((sv0, dv0, si0), (sv1, dv1, si1))

        def idx_start(g):
            sv, dv, si = idxb[g % 2]
            pltpu.async_copy(src_hbm.at[wid, g], sv, si)
            pltpu.async_copy(dst_hbm.at[wid, g], dv, si)

        def idx_wait(g):
            sv, dv, si = idxb[g % 2]
            pltpu.make_async_copy(src_hbm.at[wid, g], sv, si).wait()
            pltpu.make_async_copy(dst_hbm.at[wid, g], dv, si).wait()

        bufs = ((b0, s0), (b1, s1), (b2, s2))

        def issue(g, j, b):
            pltpu.async_copy(
                h_hbm.at[idxb[g % 2][0].at[j]], bufs[b][0], bufs[b][1])

        def proc(g, j, b):
            pltpu.make_async_copy(
                h_hbm.at[idxb[g % 2][0].at[j]], bufs[b][0], bufs[b][1]).wait()
            pltpu.sync_copy(bufs[b][0], acc.at[idxb[g % 2][1].at[j]], add=True)

        idx_start(0)

        def zrow(r, carry):
            for c in range(D // 16):
                b2[r, pl.ds(c * 16, 16)] = jnp.zeros((16,), jnp.float32)
            return carry

        lax.fori_loop(0, CH, zrow, 0)
        idx_wait(0)
        issue(0, 0, 0)
        issue(0, 1, 1)
        idx_start(1)
        for k in range(RPS // CH):
            pltpu.sync_copy(b2, acc.at[pl.ds(sid * RPS + k * CH, CH)])
        plsc.subcore_barrier()

        for g in range(NG):

            def body(i, carry, g=g):
                j = 3 * i
                issue(g, j + 2, 2)
                proc(g, j, 0)
                issue(g, j + 3, 0)
                proc(g, j + 1, 1)
                issue(g, j + 4, 1)
                proc(g, j + 2, 2)
                return carry

            lax.fori_loop(0, (CPG - 4) // 3, body, 0)
            issue(g, CPG - 2, 2)
            proc(g, CPG - 4, 0)
            issue(g, CPG - 1, 0)
            proc(g, CPG - 3, 1)
            proc(g, CPG - 2, 2)
            proc(g, CPG - 1, 0)

            if g + 1 < NG:
                idx_wait(g + 1)
                issue(g + 1, 0, 0)
                issue(g + 1, 1, 1)
                if g + 2 < NG:
                    idx_start(g + 2)

        plsc.subcore_barrier()
        pltpu.sync_copy(acc.at[pl.ds(sid * RPS, RPS)],
                        out_hbm.at[cid, pl.ds(sid * RPS, RPS)])

    return agg(h, src_r, dst_r)


BR = 5000


def _mlp_body(x_ref, a_ref, s_ref, w1_ref, b1_ref, w2_ref, b2_ref, o_ref):
    z = x_ref[...] * s_ref[...] + a_ref[0] + a_ref[1]
    z = jnp.maximum(
        jnp.dot(z, w1_ref[...], preferred_element_type=jnp.float32)
        + b1_ref[...], 0.0)
    z = jnp.maximum(
        jnp.dot(z, w2_ref[...], preferred_element_type=jnp.float32)
        + b2_ref[...], 0.0)
    o_ref[...] = z


def _mlp_tc(x, a, s, W1, b1, W2, b2):
    grid = (N // BR,)
    return pl.pallas_call(
        _mlp_body,
        grid=grid,
        in_specs=[
            pl.BlockSpec((BR, D), lambda i: (i, 0)),
            pl.BlockSpec((2, BR, D), lambda i: (0, i, 0)),
            pl.BlockSpec((1, D), lambda i: (0, 0)),
            pl.BlockSpec((D, H), lambda i: (0, 0)),
            pl.BlockSpec((1, H), lambda i: (0, 0)),
            pl.BlockSpec((H, H), lambda i: (0, 0)),
            pl.BlockSpec((1, H), lambda i: (0, 0)),
        ],
        out_specs=pl.BlockSpec((BR, H), lambda i: (i, 0)),
        out_shape=jax.ShapeDtypeStruct((N, H), jnp.float32),
    )(x, a, s, W1, b1, W2, b2)


def _mlp_pool_body(x_ref, a_ref, s_ref, w1_ref, b1_ref, w2_ref, b2_ref,
                   batch_ref, wc_ref, bc_ref, o_ref, acc_s, acc_c):
    i = pl.program_id(0)
    z = x_ref[...] * s_ref[...] + a_ref[0] + a_ref[1]
    z = jnp.maximum(
        jnp.dot(z, w1_ref[...], preferred_element_type=jnp.float32)
        + b1_ref[...], 0.0)
    h2 = jnp.maximum(
        jnp.dot(z, w2_ref[...], preferred_element_type=jnp.float32)
        + b2_ref[...], 0.0)
    b = batch_ref[0]
    gid = lax.broadcasted_iota(jnp.int32, (G, BR), 0)
    p = (gid == jnp.broadcast_to(b, (G, BR))).astype(jnp.float32)

    @pl.when(i == 0)
    def _():
        acc_s[...] = jnp.zeros_like(acc_s)
        acc_c[...] = jnp.zeros_like(acc_c)

    acc_s[...] += jnp.dot(p, h2, preferred_element_type=jnp.float32)
    acc_c[...] += jnp.broadcast_to(
        jnp.sum(p, axis=1, keepdims=True), (G, H))

    @pl.when(i == pl.num_programs(0) - 1)
    def _():
        rep = acc_s[...] / jnp.maximum(acc_c[...], 1.0)
        o_ref[...] = (
            jnp.dot(rep, wc_ref[...], preferred_element_type=jnp.float32)
            + bc_ref[...])


def _mlp_pool_tc(x, a, s, W1, b1, W2, b2, batch_r, Wc_pad, bc_pad):
    grid = (N // BR,)
    return pl.pallas_call(
        _mlp_pool_body,
        grid=grid,
        in_specs=[
            pl.BlockSpec((BR, D), lambda i: (i, 0)),
            pl.BlockSpec((2, BR, D), lambda i: (0, i, 0)),
            pl.BlockSpec((1, D), lambda i: (0, 0)),
            pl.BlockSpec((D, H), lambda i: (0, 0)),
            pl.BlockSpec((1, H), lambda i: (0, 0)),
            pl.BlockSpec((H, H), lambda i: (0, 0)),
            pl.BlockSpec((1, H), lambda i: (0, 0)),
            pl.BlockSpec((1, 1, BR), lambda i: (i, 0, 0)),
            pl.BlockSpec((H, 128), lambda i: (0, 0)),
            pl.BlockSpec((1, 128), lambda i: (0, 0)),
        ],
        out_specs=pl.BlockSpec((G, 128), lambda i: (0, 0)),
        out_shape=jax.ShapeDtypeStruct((G, 128), jnp.float32),
        scratch_shapes=[
            pltpu.VMEM((G, H), jnp.float32),
            pltpu.VMEM((G, H), jnp.float32),
        ],
    )(x, a, s, W1, b1, W2, b2, batch_r, Wc_pad, bc_pad)


def kernel(x, edge_index, batch, eps0, W1_0, b1_0, W2_0, b2_0,
           eps1, W1_1, b1_1, W2_1, b2_1, Wc, bc):
    src_r = edge_index[0].reshape(NW, NG, CPG, CH)
    dst_r = edge_index[1].reshape(NW, NG, CPG, CH)
    ones_row = jnp.ones((1, D), jnp.float32)
    s0 = ones_row * (1.0 + eps0)
    s1 = ones_row * (1.0 + eps1)
    batch_r = batch.reshape(N // BR, 1, BR)
    Wc_pad = jnp.zeros((H, 128), jnp.float32).at[:, :C].set(Wc)
    bc_pad = jnp.zeros((1, 128), jnp.float32).at[0, :C].set(bc)

    a0 = _segment_sum_sc(x, src_r, dst_r)
    h1 = _mlp_tc(x, a0, s0, W1_0, b1_0.reshape(1, H), W2_0, b2_0.reshape(1, H))
    a1 = _segment_sum_sc(h1, src_r, dst_r)
    out = _mlp_pool_tc(h1, a1, s1, W1_1, b1_1.reshape(1, H),
                       W2_1, b2_1.reshape(1, H), batch_r, Wc_pad, bc_pad)
    return out[:, :C]
